# baseline (speedup 1.0000x reference)
import functools

import jax
import jax.numpy as jnp
from jax import lax
from jax.experimental import pallas as pl
from jax.experimental.pallas import tpu as pltpu
from jax.experimental.pallas import tpu_sc as plsc

_NUM_CORES = 2
_NUM_SUBCORES = 16
_NW = _NUM_CORES * _NUM_SUBCORES
_D = 128
_GATHER_ROWS = 128


_NBUF = 4


@functools.lru_cache(maxsize=None)
def _make_lookup(n_tokens: int):
    assert n_tokens % (_NW * _GATHER_ROWS) == 0
    nb = n_tokens // (_NW * _GATHER_ROWS)
    b_per_w = nb * _GATHER_ROWS

    mesh = plsc.VectorSubcoreMesh(core_axis_name="c", subcore_axis_name="s")

    @functools.partial(
        pl.kernel,
        mesh=mesh,
        out_type=jax.ShapeDtypeStruct((n_tokens, _D), jnp.float32),
        scratch_types=[
            pltpu.VMEM((nb, _GATHER_ROWS), jnp.int32),
            pltpu.VMEM((_NBUF, _GATHER_ROWS, _D), jnp.float32),
            pltpu.SemaphoreType.DMA((_NBUF,)),
            pltpu.SemaphoreType.DMA,
        ],
    )
    def lookup(idx_hbm, table_hbm, out_hbm, idx_v, rows_v, gsem, ssem):
        wid = lax.axis_index("s") * _NUM_CORES + lax.axis_index("c")
        base = wid * b_per_w
        pltpu.sync_copy(idx_hbm.at[wid], idx_v)

        pltpu.async_copy(table_hbm.at[idx_v.at[0]], rows_v.at[0], gsem.at[0])

        def step(i, carry):
            b = i % _NBUF
            nxt = i + 1
            bn = nxt % _NBUF

            @pl.when(nxt < nb)
            def _issue_next():
                pltpu.async_copy(
                    table_hbm.at[idx_v.at[nxt]], rows_v.at[bn], gsem.at[bn]
                )

            pltpu.make_async_copy(
                table_hbm.at[idx_v.at[i]], rows_v.at[b], gsem.at[b]
            ).wait()

            @pl.when(i >= 1)
            def _drain_prev_store():
                pltpu.make_async_copy(
                    rows_v.at[b], out_hbm.at[pl.ds(base, _GATHER_ROWS)], ssem
                ).wait()

            off = base + i * _GATHER_ROWS
            pltpu.async_copy(
                rows_v.at[b], out_hbm.at[pl.ds(off, _GATHER_ROWS)], ssem
            )
            return carry

        lax.fori_loop(0, nb, step, 0)

        pltpu.make_async_copy(
            rows_v.at[0], out_hbm.at[pl.ds(base, _GATHER_ROWS)], ssem
        ).wait()

    return lookup


def kernel(token_ids, embed_mat):
    b, s = token_ids.shape
    n = b * s
    idx = token_ids.astype(jnp.int32).reshape(
        _NW, n // (_NW * _GATHER_ROWS), _GATHER_ROWS
    )
    out = _make_lookup(n)(idx, embed_mat)
    return out.reshape(b, s, _D)


# speedup vs baseline: 1.0019x; 1.0019x over previous
import functools

import jax
import jax.numpy as jnp
from jax import lax
from jax.experimental import pallas as pl
from jax.experimental.pallas import tpu as pltpu
from jax.experimental.pallas import tpu_sc as plsc

_NUM_CORES = 2
_NUM_SUBCORES = 16
_NW = _NUM_CORES * _NUM_SUBCORES
_D = 128
_GATHER_ROWS = 128


_NBUF = 6
_DEPTH = 3


@functools.lru_cache(maxsize=None)
def _make_lookup(n_tokens: int):
    assert n_tokens % (_NW * _GATHER_ROWS) == 0
    nb = n_tokens // (_NW * _GATHER_ROWS)
    b_per_w = nb * _GATHER_ROWS

    mesh = plsc.VectorSubcoreMesh(core_axis_name="c", subcore_axis_name="s")

    @functools.partial(
        pl.kernel,
        mesh=mesh,
        out_type=jax.ShapeDtypeStruct((n_tokens, _D), jnp.float32),
        scratch_types=[
            pltpu.VMEM((nb, _GATHER_ROWS), jnp.int32),
            pltpu.VMEM((_NBUF, _GATHER_ROWS, _D), jnp.float32),
            pltpu.SemaphoreType.DMA((_NBUF,)),
            pltpu.SemaphoreType.DMA,
        ],
    )
    def lookup(idx_hbm, table_hbm, out_hbm, idx_v, rows_v, gsem, ssem):
        wid = lax.axis_index("s") * _NUM_CORES + lax.axis_index("c")
        base = wid * b_per_w
        pltpu.sync_copy(idx_hbm.at[wid], idx_v)

        for j in range(_DEPTH):
            pltpu.async_copy(table_hbm.at[idx_v.at[j]], rows_v.at[j], gsem.at[j])

        def step(i, carry):
            b = i % _NBUF
            nxt = i + _DEPTH
            bn = nxt % _NBUF

            @pl.when(nxt < nb)
            def _issue_next():
                pltpu.async_copy(
                    table_hbm.at[idx_v.at[nxt]], rows_v.at[bn], gsem.at[bn]
                )

            pltpu.make_async_copy(
                table_hbm.at[idx_v.at[i]], rows_v.at[b], gsem.at[b]
            ).wait()

            @pl.when(i >= 1)
            def _drain_prev_store():
                pltpu.make_async_copy(
                    rows_v.at[b], out_hbm.at[pl.ds(base, _GATHER_ROWS)], ssem
                ).wait()

            off = base + i * _GATHER_ROWS
            pltpu.async_copy(
                rows_v.at[b], out_hbm.at[pl.ds(off, _GATHER_ROWS)], ssem
            )
            return carry

        lax.fori_loop(0, nb, step, 0)

        pltpu.make_async_copy(
            rows_v.at[0], out_hbm.at[pl.ds(base, _GATHER_ROWS)], ssem
        ).wait()

    return lookup


def kernel(token_ids, embed_mat):
    b, s = token_ids.shape
    n = b * s
    idx = token_ids.astype(jnp.int32).reshape(
        _NW, n // (_NW * _GATHER_ROWS), _GATHER_ROWS
    )
    out = _make_lookup(n)(idx, embed_mat)
    return out.reshape(b, s, _D)
